# baseline (speedup 1.0000x reference)
import functools

import jax
import jax.numpy as jnp
from jax import lax
from jax.experimental import pallas as pl
from jax.experimental.pallas import tpu as pltpu
from jax.experimental.pallas import tpu_sc as plsc

NC = 2
NS = 16
NW = NC * NS
LANES = 16

BATCH = 16384
HALF = BATCH // 2
DIM = 128
B_PER_W = HALF // NW
CHUNK = 128
N_CHUNKS = B_PER_W // CHUNK


def _sc_gather_body(idx_hbm, table_hbm, icept_hbm, grows_hbm, gbias_hbm,
                    idx_v, rows_v, bias_v, sem_t, sem_b, sem_o, sem_ob):
    wid = lax.axis_index("s") * NC + lax.axis_index("c")
    wbase = wid * B_PER_W

    for c in range(N_CHUNKS):
        pltpu.sync_copy(idx_hbm.at[pl.ds(wbase + c * CHUNK, CHUNK)],
                        idx_v[c])
    gathers = [
        pltpu.async_copy(table_hbm.at[idx_v[c]], rows_v[c], sem_t[c])
        for c in range(N_CHUNKS)
    ]
    bias_g = pltpu.async_copy(icept_hbm.at[idx_v[0]], bias_v[0], sem_b[0])
    bias_g2 = pltpu.async_copy(icept_hbm.at[idx_v[1]], bias_v[1], sem_b[1])
    outs = []
    for c in range(N_CHUNKS):
        gathers[c].wait()
        outs.append(pltpu.async_copy(
            rows_v[c], grows_hbm.at[pl.ds(wbase + c * CHUNK, CHUNK)],
            sem_o[c]))
    bias_g.wait()
    bias_g2.wait()
    ob0 = pltpu.async_copy(bias_v[0], gbias_hbm.at[pl.ds(wbase, CHUNK)],
                           sem_ob[0])
    ob1 = pltpu.async_copy(bias_v[1],
                           gbias_hbm.at[pl.ds(wbase + CHUNK, CHUNK)],
                           sem_ob[1])
    for o in outs:
        o.wait()
    ob0.wait()
    ob1.wait()


def _sc_gather(index, table, icept):
    mesh = plsc.VectorSubcoreMesh(
        core_axis_name="c", subcore_axis_name="s",
        num_cores=NC, num_subcores=NS)
    dbl = lambda t: [t, t]
    run = pl.kernel(
        _sc_gather_body,
        out_type=(
            jax.ShapeDtypeStruct((HALF, DIM), jnp.float32),
            jax.ShapeDtypeStruct((HALF,), jnp.float32),
        ),
        mesh=mesh,
        compiler_params=pltpu.CompilerParams(needs_layout_passes=False),
        scratch_types=[
            dbl(pltpu.VMEM((CHUNK,), jnp.int32)),
            dbl(pltpu.VMEM((CHUNK, DIM), jnp.float32)),
            dbl(pltpu.VMEM((CHUNK,), jnp.float32)),
            dbl(pltpu.SemaphoreType.DMA),
            dbl(pltpu.SemaphoreType.DMA),
            dbl(pltpu.SemaphoreType.DMA),
            dbl(pltpu.SemaphoreType.DMA),
        ],
    )
    return run(index, table, icept)


TC_BLOCK = 4096


def _tc_dot_body(x_ref, g_ref, b_ref, o_ref):
    prod = x_ref[...] * g_ref[...]
    ones = jnp.ones((1, DIM), jnp.float32)
    res = jax.lax.dot_general(ones, prod, (((1,), (1,)), ((), ())),
                              preferred_element_type=jnp.float32)
    o_ref[...] = (res + b_ref[0]).reshape(1, 1, TC_BLOCK)


def _tc_dot(x, grows, gbias):
    grid = HALF // TC_BLOCK
    out2d = pl.pallas_call(
        _tc_dot_body,
        grid=(grid,),
        in_specs=[
            pl.BlockSpec((TC_BLOCK, DIM), lambda i: (i, 0)),
            pl.BlockSpec((TC_BLOCK, DIM), lambda i: (i, 0)),
            pl.BlockSpec((1, 1, TC_BLOCK), lambda i: (i, 0, 0)),
        ],
        out_specs=pl.BlockSpec((1, 1, TC_BLOCK), lambda i: (i, 0, 0)),
        out_shape=jax.ShapeDtypeStruct((grid, 1, TC_BLOCK), jnp.float32),
        compiler_params=pltpu.CompilerParams(
            dimension_semantics=("arbitrary",)),
    )(x, grows, gbias.reshape(grid, 1, TC_BLOCK))
    return out2d.reshape(HALF)


@jax.jit
def _alo_forward(x, index, table, icept):
    ga, ba = _sc_gather(index[:HALF], table, icept)
    gb, bb = _sc_gather(index[HALF:], table, icept)
    oa = _tc_dot(x[:HALF], ga, ba)
    ob = _tc_dot(x[HALF:], gb, bb)
    return jnp.concatenate([oa, ob])


def kernel(x, index, treeplus_loo_layer, treeplus_loo_intercept):
    index = index.astype(jnp.int32)
    return _alo_forward(x, index, treeplus_loo_layer, treeplus_loo_intercept)


# speedup vs baseline: 1.1770x; 1.1770x over previous
import functools

import jax
import jax.numpy as jnp
from jax import lax
from jax.experimental import pallas as pl
from jax.experimental.pallas import tpu as pltpu
from jax.experimental.pallas import tpu_sc as plsc

NC = 2
NS = 16
NW = NC * NS
LANES = 16

BATCH = 16384
DIM = 128
B_PER_W = BATCH // NW
CHUNK = 128
N_CHUNKS = B_PER_W // CHUNK


def _sc_gather_body(idx_hbm, table_hbm, icept_hbm, grows_hbm, gbias_hbm,
                    idx_v, rows_v, bias_v, sem_t, sem_b, sem_o, sem_ob):
    wid = lax.axis_index("s") * NC + lax.axis_index("c")
    wbase = wid * B_PER_W

    for c in range(N_CHUNKS):
        pltpu.sync_copy(idx_hbm.at[pl.ds(wbase + c * CHUNK, CHUNK)],
                        idx_v[c])
    gathers = [
        pltpu.async_copy(table_hbm.at[idx_v[c]], rows_v[c], sem_t[c])
        for c in range(N_CHUNKS)
    ]
    bias_gs = [
        pltpu.async_copy(icept_hbm.at[idx_v[c]], bias_v[c], sem_b[c])
        for c in range(N_CHUNKS)
    ]
    outs = []
    for c in range(N_CHUNKS):
        gathers[c].wait()
        outs.append(pltpu.async_copy(
            rows_v[c], grows_hbm.at[pl.ds(wbase + c * CHUNK, CHUNK)],
            sem_o[c]))
    obs = []
    for c in range(N_CHUNKS):
        bias_gs[c].wait()
        obs.append(pltpu.async_copy(
            bias_v[c], gbias_hbm.at[pl.ds(wbase + c * CHUNK, CHUNK)],
            sem_ob[c]))
    for o in outs:
        o.wait()
    for o in obs:
        o.wait()


def _sc_gather(index, table, icept):
    mesh = plsc.VectorSubcoreMesh(
        core_axis_name="c", subcore_axis_name="s",
        num_cores=NC, num_subcores=NS)
    dbl = lambda t: [t] * N_CHUNKS
    run = pl.kernel(
        _sc_gather_body,
        out_type=(
            jax.ShapeDtypeStruct((BATCH, DIM), jnp.float32),
            jax.ShapeDtypeStruct((BATCH,), jnp.float32),
        ),
        mesh=mesh,
        compiler_params=pltpu.CompilerParams(needs_layout_passes=False),
        scratch_types=[
            dbl(pltpu.VMEM((CHUNK,), jnp.int32)),
            dbl(pltpu.VMEM((CHUNK, DIM), jnp.float32)),
            dbl(pltpu.VMEM((CHUNK,), jnp.float32)),
            dbl(pltpu.SemaphoreType.DMA),
            dbl(pltpu.SemaphoreType.DMA),
            dbl(pltpu.SemaphoreType.DMA),
            dbl(pltpu.SemaphoreType.DMA),
        ],
    )
    return run(index, table, icept)


TC_BLOCK = 8192


def _tc_dot_body(x_ref, g_ref, b_ref, o_ref):
    prod = x_ref[...] * g_ref[...]
    ones = jnp.ones((1, DIM), jnp.float32)
    res = jax.lax.dot_general(ones, prod, (((1,), (1,)), ((), ())),
                              preferred_element_type=jnp.float32)
    o_ref[...] = (res + b_ref[0]).reshape(1, 1, TC_BLOCK)


def _tc_dot(x, grows, gbias):
    grid = BATCH // TC_BLOCK
    out2d = pl.pallas_call(
        _tc_dot_body,
        grid=(grid,),
        in_specs=[
            pl.BlockSpec((TC_BLOCK, DIM), lambda i: (i, 0)),
            pl.BlockSpec((TC_BLOCK, DIM), lambda i: (i, 0)),
            pl.BlockSpec((1, 1, TC_BLOCK), lambda i: (i, 0, 0)),
        ],
        out_specs=pl.BlockSpec((1, 1, TC_BLOCK), lambda i: (i, 0, 0)),
        out_shape=jax.ShapeDtypeStruct((grid, 1, TC_BLOCK), jnp.float32),
        compiler_params=pltpu.CompilerParams(
            dimension_semantics=("arbitrary",)),
    )(x, grows, gbias.reshape(grid, 1, TC_BLOCK))
    return out2d.reshape(BATCH)


@jax.jit
def _alo_forward(x, index, table, icept):
    grows, gbias = _sc_gather(index, table, icept)
    return _tc_dot(x, grows, gbias)


def kernel(x, index, treeplus_loo_layer, treeplus_loo_intercept):
    index = index.astype(jnp.int32)
    return _alo_forward(x, index, treeplus_loo_layer, treeplus_loo_intercept)


# speedup vs baseline: 1.2172x; 1.0342x over previous
import functools

import jax
import jax.numpy as jnp
from jax import lax
from jax.experimental import pallas as pl
from jax.experimental.pallas import tpu as pltpu
from jax.experimental.pallas import tpu_sc as plsc

NC = 2
NS = 16
NW = NC * NS
LANES = 16

BATCH = 16384
DIM = 128
B_PER_W = BATCH // NW
CHUNK = 256
N_CHUNKS = B_PER_W // CHUNK


def _sc_gather_body(idx_hbm, table_hbm, icept_hbm, grows_hbm, gbias_hbm,
                    idx_v, rows_v, bias_v, sem_t, sem_b, sem_o, sem_ob):
    wid = lax.axis_index("s") * NC + lax.axis_index("c")
    wbase = wid * B_PER_W

    for c in range(N_CHUNKS):
        pltpu.sync_copy(idx_hbm.at[pl.ds(wbase + c * CHUNK, CHUNK)],
                        idx_v[c])
    gathers = [
        pltpu.async_copy(table_hbm.at[idx_v[c]], rows_v[c], sem_t[c])
        for c in range(N_CHUNKS)
    ]
    bias_g = pltpu.async_copy(icept_hbm.at[idx_v[0]], bias_v[0], sem_b[0])
    bias_g2 = pltpu.async_copy(icept_hbm.at[idx_v[1]], bias_v[1], sem_b[1])
    outs = []
    for c in range(N_CHUNKS):
        gathers[c].wait()
        outs.append(pltpu.async_copy(
            rows_v[c], grows_hbm.at[pl.ds(wbase + c * CHUNK, CHUNK)],
            sem_o[c]))
    bias_g.wait()
    bias_g2.wait()
    ob0 = pltpu.async_copy(bias_v[0], gbias_hbm.at[pl.ds(wbase, CHUNK)],
                           sem_ob[0])
    ob1 = pltpu.async_copy(bias_v[1],
                           gbias_hbm.at[pl.ds(wbase + CHUNK, CHUNK)],
                           sem_ob[1])
    for o in outs:
        o.wait()
    ob0.wait()
    ob1.wait()


def _sc_gather(index, table, icept):
    mesh = plsc.VectorSubcoreMesh(
        core_axis_name="c", subcore_axis_name="s",
        num_cores=NC, num_subcores=NS)
    dbl = lambda t: [t, t]
    run = pl.kernel(
        _sc_gather_body,
        out_type=(
            jax.ShapeDtypeStruct((BATCH, DIM), jnp.float32),
            jax.ShapeDtypeStruct((BATCH,), jnp.float32),
        ),
        mesh=mesh,
        compiler_params=pltpu.CompilerParams(needs_layout_passes=False),
        scratch_types=[
            dbl(pltpu.VMEM((CHUNK,), jnp.int32)),
            dbl(pltpu.VMEM((CHUNK, DIM), jnp.float32)),
            dbl(pltpu.VMEM((CHUNK,), jnp.float32)),
            dbl(pltpu.SemaphoreType.DMA),
            dbl(pltpu.SemaphoreType.DMA),
            dbl(pltpu.SemaphoreType.DMA),
            dbl(pltpu.SemaphoreType.DMA),
        ],
    )
    return run(index, table, icept)


TC_BLOCK = 8192


def _tc_dot_body(x_ref, g_ref, b_ref, o_ref):
    prod = x_ref[...] * g_ref[...]
    ones = jnp.ones((1, DIM), jnp.float32)
    res = jax.lax.dot_general(ones, prod, (((1,), (1,)), ((), ())),
                              preferred_element_type=jnp.float32)
    o_ref[...] = (res + b_ref[0]).reshape(1, 1, TC_BLOCK)


def _tc_dot(x, grows, gbias):
    grid = BATCH // TC_BLOCK
    out2d = pl.pallas_call(
        _tc_dot_body,
        grid=(grid,),
        in_specs=[
            pl.BlockSpec((TC_BLOCK, DIM), lambda i: (i, 0)),
            pl.BlockSpec((TC_BLOCK, DIM), lambda i: (i, 0)),
            pl.BlockSpec((1, 1, TC_BLOCK), lambda i: (i, 0, 0)),
        ],
        out_specs=pl.BlockSpec((1, 1, TC_BLOCK), lambda i: (i, 0, 0)),
        out_shape=jax.ShapeDtypeStruct((grid, 1, TC_BLOCK), jnp.float32),
        compiler_params=pltpu.CompilerParams(
            dimension_semantics=("arbitrary",)),
    )(x, grows, gbias.reshape(grid, 1, TC_BLOCK))
    return out2d.reshape(BATCH)


@jax.jit
def _alo_forward(x, index, table, icept):
    grows, gbias = _sc_gather(index, table, icept)
    return _tc_dot(x, grows, gbias)


def kernel(x, index, treeplus_loo_layer, treeplus_loo_intercept):
    index = index.astype(jnp.int32)
    return _alo_forward(x, index, treeplus_loo_layer, treeplus_loo_intercept)
